# Initial kernel scaffold; baseline (speedup 1.0000x reference)
import jax
import jax.numpy as jnp
from jax.experimental import pallas as pl


def kernel(features_1, features_2, edge_index_1, edge_index_2, W1, b1, W2, b2, W3, b3, Wl, bl, Wr, br, att, bias_g, Wa, Wt, Wblock, bt, Wf, bf, Ws, bs):
    raise NotImplementedError("write your pallas kernel here")


# speedup vs baseline: 1.6033x; 1.6033x over previous
import functools

import jax
import jax.numpy as jnp
import numpy as np
from jax.experimental import pallas as pl

N = 10000
D = 128
F1, F2, F3, F4 = 128, 64, 32, 32
TN = 16
BN = 16


def _mask_scale(key, p, shape):
    keep = 1.0 - p
    b = jax.random.bernoulli(key, keep, shape)
    return jnp.where(b, jnp.float32(1.0) / jnp.float32(keep), jnp.float32(0.0))


def _ew_kernel(x_ref, m_ref, b_ref, dis_ref, o_ref):
    z = jax.nn.relu(x_ref[...] * dis_ref[...] + b_ref[...])
    o_ref[...] = z * m_ref[...]


def _ew_apply(x, m, b, dis):
    n, f = x.shape
    blk = 1000
    return pl.pallas_call(
        _ew_kernel,
        out_shape=jax.ShapeDtypeStruct((n, f), jnp.float32),
        grid=(n // blk,),
        in_specs=[
            pl.BlockSpec((blk, f), lambda i: (i, 0)),
            pl.BlockSpec((blk, f), lambda i: (i, 0)),
            pl.BlockSpec((1, f), lambda i: (0, 0)),
            pl.BlockSpec((blk, 1), lambda i: (i, 0)),
        ],
        out_specs=pl.BlockSpec((blk, f), lambda i: (i, 0)),
    )(x, m, b.reshape(1, f), dis.reshape(n, 1))


def _gcn_agg(hhat, s, d, dis, b, m):
    agg = jax.ops.segment_sum(hhat[s], d, num_segments=N)
    return _ew_apply(agg, m, b, dis)


def _gatv2(x, s, d, Wl, bl, Wr, br, att, bias_g):
    xl = x @ Wl + bl
    xr = x @ Wr + br
    mhat = jax.nn.leaky_relu(xl + xr, negative_slope=0.2) @ att
    e = jax.nn.leaky_relu(xl[s] + xr[d], negative_slope=0.2)
    alpha = e @ att
    a = jnp.exp(alpha - mhat[d])
    denom = jax.ops.segment_sum(a, d, num_segments=N)
    acc = jax.ops.segment_sum(a[:, None] * xl[s], d, num_segments=N)
    return acc / (denom[:, None] + 1e-16) + bias_g


def _conv_pass(x, ei, params, tag):
    W1, b1, W2, b2, W3, b3, Wl, bl, Wr, br, att, bias_g = params
    src, dst = ei[0], ei[1]
    loop = jnp.arange(N)
    s = jnp.concatenate([src, loop])
    d = jnp.concatenate([dst, loop])
    deg = jax.ops.segment_sum(jnp.ones(s.shape[0], jnp.float32), d, num_segments=N)
    dis = jnp.where(deg > 0, jax.lax.rsqrt(deg), 0.0)

    base = jax.random.key(1234 + tag)
    m1 = _mask_scale(jax.random.fold_in(base, 0), 0.8, (N, F1))
    m2 = _mask_scale(jax.random.fold_in(base, 1), 0.5, (N, F2))
    m3 = _mask_scale(jax.random.fold_in(base, 2), 0.5, (N, F3))

    dx = dis[:, None]
    h = _gcn_agg((dx * x) @ W1, s, d, dis, b1, m1)
    h = _gcn_agg((dx * h) @ W2, s, d, dis, b2, m2)
    h = _gcn_agg((dx * h) @ W3, s, d, dis, b3, m3)
    return _gatv2(h, s, d, Wl, bl, Wr, br, att, bias_g)


def _attention_pool(x, Wa):
    g = jnp.tanh((jnp.mean(x, axis=0) @ Wa))
    sig = jax.nn.sigmoid(x @ g)
    return x.T @ sig[:, None]


def _tensor_net(e1, e2, Wt, Wblock, bt):
    scoring = (e1.T @ Wt.reshape(F4, F4 * TN)).reshape(F4, TN)
    scoring = scoring.T @ e2
    combined = jnp.concatenate([e1, e2], axis=0)
    block = Wblock @ combined
    return jax.nn.relu(scoring + block + bt)


def kernel(features_1, features_2, edge_index_1, edge_index_2, W1, b1, W2, b2, W3, b3, Wl, bl, Wr, br, att, bias_g, Wa, Wt, Wblock, bt, Wf, bf, Ws, bs):
    params = (W1, b1, W2, b2, W3, b3, Wl, bl, Wr, br, att, bias_g)
    a1 = _conv_pass(features_1, edge_index_1, params, 1)
    a2 = _conv_pass(features_2, edge_index_2, params, 2)
    p1 = _attention_pool(a1, Wa)
    p2 = _attention_pool(a2, Wa)
    scores = _tensor_net(p1, p2, Wt, Wblock, bt).T
    h = scores @ Wf.T + bf
    nrm = jnp.maximum(jnp.linalg.norm(h, axis=1, keepdims=True), 1e-12)
    h = h / nrm
    return jax.nn.relu(h @ Ws.T + bs)


# speedup vs baseline: 2.6627x; 1.6608x over previous
import functools

import jax
import jax.numpy as jnp
import numpy as np
from jax import lax
from jax.experimental import pallas as pl
from jax.experimental.pallas import tpu as pltpu
from jax.experimental.pallas import tpu_sc as plsc

N = 10000
E = 320000
D = 128
F1, F2, F3, F4 = 128, 64, 32, 32
TN = 16
BN = 16

NC, NS, LN = 2, 16, 16
W = 128
NWIN = 168
EPT = NWIN * W
EA_PAD = NS * EPT
EA = E + N
PAD = EA_PAD - EA
NROW = 10240
NBUF = 3

_mesh = plsc.VectorSubcoreMesh(core_axis_name="c", subcore_axis_name="s")


def _zero_rows(buf, nrow, f):
    z = jnp.zeros((LN,), jnp.float32)

    def body(r, _):
        for jj in range(f // LN):
            buf[0, r, pl.ds(jj * LN, LN)] = z
        return 0

    lax.fori_loop(0, nrow, body, 0, unroll=False)


def _sc_agg(f):
    spt = NROW // NS
    if f == 128:
        nbuf, slots, ch = 2, 2, 8
    else:
        nbuf, slots, ch = 3, 1, NWIN

    kw = {}
    if f != 128:
        kw["compiler_params"] = pltpu.CompilerParams(use_tc_tiling_on_sc=False)

    @functools.partial(
        pl.kernel,
        out_type=jax.ShapeDtypeStruct((NC, NROW, f), jnp.float32),
        mesh=_mesh,
        scratch_types=[
            pltpu.VMEM((slots, ch, W), jnp.int32),
            pltpu.VMEM((slots, ch, W), jnp.int32),
            pltpu.VMEM((nbuf, W, f), jnp.float32),
            pltpu.VMEM_SHARED((NROW, f), jnp.float32),
            pltpu.SemaphoreType.DMA((nbuf,)),
        ],
        **kw,
    )
    def k(tbl, s_hbm, d_hbm, out, sidx, didx, rows, acc, sem):
        c = lax.axis_index("c")
        t = lax.axis_index("s")
        pltpu.sync_copy(s_hbm.at[c, t, pl.ds(0, ch)], sidx.at[0])
        pltpu.sync_copy(d_hbm.at[c, t, pl.ds(0, ch)], didx.at[0])
        _zero_rows(rows, W, f)
        for kk in range(spt // W):
            pltpu.sync_copy(rows.at[0], acc.at[pl.ds(t * spt + kk * W, W)])
        plsc.subcore_barrier()

        def gref(w, b):
            slot = (w // ch) % slots
            return pltpu.make_async_copy(
                tbl.at[sidx.at[slot, w % ch]], rows.at[b], sem.at[b])

        for b in range(nbuf):
            gref(b, b).start()

        def body(i, _):
            for b in range(nbuf):
                w = i * nbuf + b
                gref(w, b).wait()
                slot = (w // ch) % slots
                pltpu.sync_copy(rows.at[b], acc.at[didx.at[slot, w % ch]],
                                add=True)
                wn = w + nbuf

                @pl.when(jnp.logical_and(wn < NWIN, wn % ch == 0))
                def _():
                    nc_ = wn // ch
                    ns_ = nc_ % slots
                    pltpu.sync_copy(s_hbm.at[c, t, pl.ds(nc_ * ch, ch)],
                                    sidx.at[ns_])
                    pltpu.sync_copy(d_hbm.at[c, t, pl.ds(nc_ * ch, ch)],
                                    didx.at[ns_])

                @pl.when(wn < NWIN)
                def _():
                    gref(wn, b).start()

            return 0

        lax.fori_loop(0, NWIN // nbuf, body, 0, unroll=False)
        plsc.subcore_barrier()
        pltpu.sync_copy(acc.at[pl.ds(t * spt, spt)], out.at[c, pl.ds(t * spt, spt)])

    return k


def _sc_deg():
    spt = NROW // NS

    @functools.partial(
        pl.kernel,
        out_type=jax.ShapeDtypeStruct((NC, NROW), jnp.float32),
        mesh=_mesh,
        scratch_types=[
            pltpu.VMEM((NWIN, W), jnp.int32),
            pltpu.VMEM((W,), jnp.float32),
            pltpu.VMEM_SHARED((NROW,), jnp.float32),
        ],
    )
    def k(d_hbm, out, didx, ones, acc):
        c = lax.axis_index("c")
        t = lax.axis_index("s")
        pltpu.sync_copy(d_hbm.at[c, t], didx)
        z = jnp.zeros((LN,), jnp.float32)
        for jj in range(W // LN):
            ones[pl.ds(jj * LN, LN)] = z
        for kk in range(spt // W):
            pltpu.sync_copy(ones, acc.at[pl.ds(t * spt + kk * W, W)])
        o = jnp.ones((LN,), jnp.float32)
        for jj in range(W // LN):
            ones[pl.ds(jj * LN, LN)] = o
        plsc.subcore_barrier()

        def body(w, _):
            pltpu.sync_copy(ones, acc.at[didx.at[w]], add=True)
            return 0

        lax.fori_loop(0, NWIN, body, 0, unroll=False)
        plsc.subcore_barrier()
        pltpu.sync_copy(acc.at[pl.ds(t * spt, spt)], out.at[c, pl.ds(t * spt, spt)])

    return k


def _mask_scale(key, p, shape):
    keep = 1.0 - p
    b = jax.random.bernoulli(key, keep, shape)
    return jnp.where(b, jnp.float32(1.0) / jnp.float32(keep), jnp.float32(0.0))


def _ew_kernel(x_ref, m_ref, b_ref, dis_ref, o_ref):
    z = jax.nn.relu(x_ref[...] * dis_ref[...] + b_ref[...])
    o_ref[...] = z * m_ref[...]


def _ew_apply(x, m, b, dis):
    n, f = x.shape
    blk = 1000
    return pl.pallas_call(
        _ew_kernel,
        out_shape=jax.ShapeDtypeStruct((n, f), jnp.float32),
        grid=(n // blk,),
        in_specs=[
            pl.BlockSpec((blk, f), lambda i: (i, 0)),
            pl.BlockSpec((blk, f), lambda i: (i, 0)),
            pl.BlockSpec((1, f), lambda i: (0, 0)),
            pl.BlockSpec((blk, 1), lambda i: (i, 0)),
        ],
        out_specs=pl.BlockSpec((blk, f), lambda i: (i, 0)),
    )(x, m, b.reshape(1, f), dis.reshape(n, 1))


def _gatv2(x, s, d, Wl, bl, Wr, br, att, bias_g):
    xl = x @ Wl + bl
    xr = x @ Wr + br
    mhat = jax.nn.leaky_relu(xl + xr, negative_slope=0.2) @ att
    e = jax.nn.leaky_relu(xl[s] + xr[d], negative_slope=0.2)
    alpha = e @ att
    a = jnp.exp(alpha - mhat[d])
    denom = jax.ops.segment_sum(a, d, num_segments=N)
    acc = jax.ops.segment_sum(a[:, None] * xl[s], d, num_segments=N)
    return acc / (denom[:, None] + 1e-16) + bias_g


def _attention_pool(x, Wa):
    g = jnp.tanh(jnp.mean(x, axis=0) @ Wa)
    sig = jax.nn.sigmoid(x @ g)
    return x.T @ sig[:, None]


def _tensor_net(e1, e2, Wt, Wblock, bt):
    scoring = (e1.T @ Wt.reshape(F4, F4 * TN)).reshape(F4, TN)
    scoring = scoring.T @ e2
    combined = jnp.concatenate([e1, e2], axis=0)
    block = Wblock @ combined
    return jax.nn.relu(scoring + block + bt)


def kernel(features_1, features_2, edge_index_1, edge_index_2, W1, b1, W2, b2, W3, b3, Wl, bl, Wr, br, att, bias_g, Wa, Wt, Wblock, bt, Wf, bf, Ws, bs):
    loop = jnp.arange(N, dtype=jnp.int32)
    padi = (jnp.arange(PAD, dtype=jnp.int32) % 64)

    def edges(ei, off):
        s = jnp.concatenate([ei[0].astype(jnp.int32), loop, padi]) + off
        d = jnp.concatenate([ei[1].astype(jnp.int32), loop, padi + N])
        return s, d

    s1, d1 = edges(edge_index_1, 0)
    s2, d2 = edges(edge_index_2, N)
    s_r = jnp.stack([s1, s2]).reshape(NC, NS, NWIN, W)
    d_r = jnp.stack([d1, d2]).reshape(NC, NS, NWIN, W)

    degp = _sc_deg()(d_r)
    deg = degp[:, :N]
    dis = lax.rsqrt(deg)

    base1 = jax.random.key(1234 + 1)
    base2 = jax.random.key(1234 + 2)
    m1 = jnp.stack([_mask_scale(jax.random.fold_in(base1, 0), 0.8, (N, F1)),
                    _mask_scale(jax.random.fold_in(base2, 0), 0.8, (N, F1))])
    m2 = jnp.stack([_mask_scale(jax.random.fold_in(base1, 1), 0.5, (N, F2)),
                    _mask_scale(jax.random.fold_in(base2, 1), 0.5, (N, F2))])
    m3 = jnp.stack([_mask_scale(jax.random.fold_in(base1, 2), 0.5, (N, F3)),
                    _mask_scale(jax.random.fold_in(base2, 2), 0.5, (N, F3))])

    x = jnp.stack([features_1, features_2])
    dx = dis[:, :, None]

    def layer(h, Wk, bk, mk, f):
        tbl = (h * dx) @ Wk
        tbl = tbl.reshape(2 * N, f)
        p = _sc_agg(f)(tbl, s_r, d_r)[:, :N, :]
        z1 = _ew_apply(p[0], mk[0], bk, dis[0])
        z2 = _ew_apply(p[1], mk[1], bk, dis[1])
        return jnp.stack([z1, z2])

    h = layer(x, W1, b1, m1, F1)
    h = layer(h, W2, b2, m2, F2)
    h = layer(h, W3, b3, m3, F3)

    a1 = _gatv2(h[0], s1[:EA], d1[:EA], Wl, bl, Wr, br, att, bias_g)
    a2 = _gatv2(h[1], s2[:EA] - N, d2[:EA], Wl, bl, Wr, br, att, bias_g)

    p1 = _attention_pool(a1, Wa)
    p2 = _attention_pool(a2, Wa)
    scores = _tensor_net(p1, p2, Wt, Wblock, bt).T
    hh = scores @ Wf.T + bf
    nrm = jnp.maximum(jnp.linalg.norm(hh, axis=1, keepdims=True), 1e-12)
    hh = hh / nrm
    return jax.nn.relu(hh @ Ws.T + bs)


# speedup vs baseline: 14.2762x; 5.3614x over previous
import functools

import jax
import jax.numpy as jnp
import numpy as np
from jax import lax
from jax.experimental import pallas as pl
from jax.experimental.pallas import tpu as pltpu
from jax.experimental.pallas import tpu_sc as plsc

N = 10000
E = 320000
D = 128
F1, F2, F3, F4 = 128, 64, 32, 32
TN = 16
BN = 16

NC, NS, LN = 2, 16, 16
W = 128
NWIN = 168
EPT = NWIN * W
EA_PAD = NS * EPT
EA = E + N
PAD = EA_PAD - EA
NROW = 10240
NBUF = 3

_mesh = plsc.VectorSubcoreMesh(core_axis_name="c", subcore_axis_name="s")


def _zero_rows(buf, nrow, f):
    z = jnp.zeros((LN,), jnp.float32)

    def body(r, _):
        for jj in range(f // LN):
            buf[0, r, pl.ds(jj * LN, LN)] = z
        return 0

    lax.fori_loop(0, nrow, body, 0, unroll=False)


def _sc_agg(f):
    spt = NROW // NS
    if f == 128:
        nbuf, slots, ch = 2, 2, 8
    else:
        nbuf, slots, ch = 3, 1, NWIN

    kw = {}
    if f != 128:
        kw["compiler_params"] = pltpu.CompilerParams(use_tc_tiling_on_sc=False)

    @functools.partial(
        pl.kernel,
        out_type=jax.ShapeDtypeStruct((NC, NROW, f), jnp.float32),
        mesh=_mesh,
        scratch_types=[
            pltpu.VMEM((slots, ch, W), jnp.int32),
            pltpu.VMEM((slots, ch, W), jnp.int32),
            pltpu.VMEM((nbuf, W, f), jnp.float32),
            pltpu.VMEM_SHARED((NROW, f), jnp.float32),
            pltpu.SemaphoreType.DMA((nbuf,)),
        ],
        **kw,
    )
    def k(tbl, s_hbm, d_hbm, out, sidx, didx, rows, acc, sem):
        c = lax.axis_index("c")
        t = lax.axis_index("s")
        pltpu.sync_copy(s_hbm.at[c, t, pl.ds(0, ch)], sidx.at[0])
        pltpu.sync_copy(d_hbm.at[c, t, pl.ds(0, ch)], didx.at[0])
        _zero_rows(rows, W, f)
        for kk in range(spt // W):
            pltpu.sync_copy(rows.at[0], acc.at[pl.ds(t * spt + kk * W, W)])
        plsc.subcore_barrier()

        def gref(w, b):
            slot = (w // ch) % slots
            return pltpu.make_async_copy(
                tbl.at[sidx.at[slot, w % ch]], rows.at[b], sem.at[b])

        for b in range(nbuf):
            gref(b, b).start()

        def body(i, _):
            for b in range(nbuf):
                w = i * nbuf + b
                gref(w, b).wait()
                slot = (w // ch) % slots
                pltpu.sync_copy(rows.at[b], acc.at[didx.at[slot, w % ch]],
                                add=True)
                wn = w + nbuf

                @pl.when(jnp.logical_and(wn < NWIN, wn % ch == 0))
                def _():
                    nc_ = wn // ch
                    ns_ = nc_ % slots
                    pltpu.sync_copy(s_hbm.at[c, t, pl.ds(nc_ * ch, ch)],
                                    sidx.at[ns_])
                    pltpu.sync_copy(d_hbm.at[c, t, pl.ds(nc_ * ch, ch)],
                                    didx.at[ns_])

                @pl.when(wn < NWIN)
                def _():
                    gref(wn, b).start()

            return 0

        lax.fori_loop(0, NWIN // nbuf, body, 0, unroll=False)
        plsc.subcore_barrier()
        pltpu.sync_copy(acc.at[pl.ds(t * spt, spt)], out.at[c, pl.ds(t * spt, spt)])

    return k


def _sc_deg():
    spt = NROW // NS

    @functools.partial(
        pl.kernel,
        out_type=jax.ShapeDtypeStruct((NC, NROW), jnp.float32),
        mesh=_mesh,
        scratch_types=[
            pltpu.VMEM((NWIN, W), jnp.int32),
            pltpu.VMEM((W,), jnp.float32),
            pltpu.VMEM_SHARED((NROW,), jnp.float32),
        ],
    )
    def k(d_hbm, out, didx, ones, acc):
        c = lax.axis_index("c")
        t = lax.axis_index("s")
        pltpu.sync_copy(d_hbm.at[c, t], didx)
        z = jnp.zeros((LN,), jnp.float32)
        for jj in range(W // LN):
            ones[pl.ds(jj * LN, LN)] = z
        for kk in range(spt // W):
            pltpu.sync_copy(ones, acc.at[pl.ds(t * spt + kk * W, W)])
        o = jnp.ones((LN,), jnp.float32)
        for jj in range(W // LN):
            ones[pl.ds(jj * LN, LN)] = o
        plsc.subcore_barrier()

        def body(w, _):
            pltpu.sync_copy(ones, acc.at[didx.at[w]], add=True)
            return 0

        lax.fori_loop(0, NWIN, body, 0, unroll=False)
        plsc.subcore_barrier()
        pltpu.sync_copy(acc.at[pl.ds(t * spt, spt)], out.at[c, pl.ds(t * spt, spt)])

    return k


FG = 48


def _sc_gat_gather():
    nbuf = 2

    @functools.partial(
        pl.kernel,
        out_type=(jax.ShapeDtypeStruct((NC, EA_PAD, F4), jnp.float32),
                  jax.ShapeDtypeStruct((NC, EA_PAD, FG), jnp.float32)),
        mesh=_mesh,
        scratch_types=[
            pltpu.VMEM((NWIN, W), jnp.int32),
            pltpu.VMEM((NWIN, W), jnp.int32),
            pltpu.VMEM((nbuf, W, F4), jnp.float32),
            pltpu.VMEM((nbuf, W, FG), jnp.float32),
            pltpu.SemaphoreType.DMA((nbuf,)),
            pltpu.SemaphoreType.DMA((nbuf,)),
            pltpu.SemaphoreType.DMA((nbuf,)),
            pltpu.SemaphoreType.DMA((nbuf,)),
        ],
        compiler_params=pltpu.CompilerParams(use_tc_tiling_on_sc=False),
    )
    def k(xlt, xrt, s_hbm, dt_hbm, oxl, oxr, sidx, didx, xbuf, rbuf,
          gsx, gsr, wsx, wsr):
        c = lax.axis_index("c")
        t = lax.axis_index("s")
        pltpu.sync_copy(s_hbm.at[c, t], sidx)
        pltpu.sync_copy(dt_hbm.at[c, t], didx)

        def grefs(w, b):
            return (pltpu.make_async_copy(xlt.at[sidx.at[w]], xbuf.at[b],
                                          gsx.at[b]),
                    pltpu.make_async_copy(xrt.at[didx.at[w]], rbuf.at[b],
                                          gsr.at[b]))

        def wrefs(w, b):
            o = t * EPT + w * W
            return (pltpu.make_async_copy(xbuf.at[b], oxl.at[c, pl.ds(o, W)],
                                          wsx.at[b]),
                    pltpu.make_async_copy(rbuf.at[b], oxr.at[c, pl.ds(o, W)],
                                          wsr.at[b]))

        for b in range(nbuf):
            for g in grefs(b, b):
                g.start()

        def body(i, _):
            for b in range(nbuf):
                w = i * nbuf + b
                for g in grefs(w, b):
                    g.wait()
                for wr in wrefs(w, b):
                    wr.start()
                wn = w + nbuf

                @pl.when(wn < NWIN)
                def _():
                    for wr in wrefs(w, b):
                        wr.wait()
                    for g in grefs(wn, b):
                        g.start()

            return 0

        lax.fori_loop(0, NWIN // nbuf, body, 0, unroll=False)
        for b in range(nbuf):
            w = NWIN - nbuf + b
            for wr in wrefs(w, b):
                wr.wait()

    return k


def _sc_gat_scatter():
    nbuf = 3
    spt = NROW // NS

    @functools.partial(
        pl.kernel,
        out_type=jax.ShapeDtypeStruct((NC, NROW, FG), jnp.float32),
        mesh=_mesh,
        scratch_types=[
            pltpu.VMEM((NWIN, W), jnp.int32),
            pltpu.VMEM((nbuf, W, FG), jnp.float32),
            pltpu.VMEM_SHARED((NROW, FG), jnp.float32),
            pltpu.SemaphoreType.DMA((nbuf,)),
        ],
        compiler_params=pltpu.CompilerParams(use_tc_tiling_on_sc=False),
    )
    def k(y_hbm, d_hbm, out, didx, rows, acc, sem):
        c = lax.axis_index("c")
        t = lax.axis_index("s")
        pltpu.sync_copy(d_hbm.at[c, t], didx)
        _zero_rows(rows, W, FG)
        for kk in range(spt // W):
            pltpu.sync_copy(rows.at[0], acc.at[pl.ds(t * spt + kk * W, W)])
        plsc.subcore_barrier()

        def gref(w, b):
            return pltpu.make_async_copy(
                y_hbm.at[c, pl.ds(t * EPT + w * W, W)], rows.at[b], sem.at[b])

        for b in range(nbuf):
            gref(b, b).start()

        def body(i, _):
            for b in range(nbuf):
                w = i * nbuf + b
                gref(w, b).wait()
                pltpu.sync_copy(rows.at[b], acc.at[didx.at[w]], add=True)
                wn = w + nbuf

                @pl.when(wn < NWIN)
                def _():
                    gref(wn, b).start()

            return 0

        lax.fori_loop(0, NWIN // nbuf, body, 0, unroll=False)
        plsc.subcore_barrier()
        pltpu.sync_copy(acc.at[pl.ds(t * spt, spt)], out.at[c, pl.ds(t * spt, spt)])

    return k


def _edge_kernel(xl_ref, xr_ref, att_ref, o_ref):
    xl = xl_ref[0]
    xrm = xr_ref[0]
    xr = xrm[:, :F4]
    mh = xrm[:, F4:F4 + 1]
    tt = xl + xr
    lr = jnp.maximum(tt, 0.2 * tt)
    al = lax.dot_general(lr, att_ref[...], (((1,), (1,)), ((), ())),
                         preferred_element_type=jnp.float32)
    a = jnp.exp(al - mh)
    y = a * xl
    blk = xl.shape[0]
    o_ref[0] = jnp.concatenate([y, a, jnp.zeros((blk, FG - F4 - 1), jnp.float32)],
                               axis=1)


def _edge_apply(XLs, XRm, att):
    blk = 4096
    nb = EA_PAD // blk
    return pl.pallas_call(
        _edge_kernel,
        out_shape=jax.ShapeDtypeStruct((NC, EA_PAD, FG), jnp.float32),
        grid=(NC, nb),
        in_specs=[
            pl.BlockSpec((1, blk, F4), lambda c, i: (c, i, 0)),
            pl.BlockSpec((1, blk, FG), lambda c, i: (c, i, 0)),
            pl.BlockSpec((1, F4), lambda c, i: (0, 0)),
        ],
        out_specs=pl.BlockSpec((1, blk, FG), lambda c, i: (c, i, 0)),
    )(XLs, XRm, att.reshape(1, F4))


def _mask_scale(key, p, shape):
    keep = 1.0 - p
    b = jax.random.bernoulli(key, keep, shape)
    return jnp.where(b, jnp.float32(1.0) / jnp.float32(keep), jnp.float32(0.0))


def _ew_kernel(x_ref, m_ref, b_ref, dis_ref, o_ref):
    z = jax.nn.relu(x_ref[...] * dis_ref[...] + b_ref[...])
    o_ref[...] = z * m_ref[...]


def _ew_apply(x, m, b, dis):
    n, f = x.shape
    blk = 1000
    return pl.pallas_call(
        _ew_kernel,
        out_shape=jax.ShapeDtypeStruct((n, f), jnp.float32),
        grid=(n // blk,),
        in_specs=[
            pl.BlockSpec((blk, f), lambda i: (i, 0)),
            pl.BlockSpec((blk, f), lambda i: (i, 0)),
            pl.BlockSpec((1, f), lambda i: (0, 0)),
            pl.BlockSpec((blk, 1), lambda i: (i, 0)),
        ],
        out_specs=pl.BlockSpec((blk, f), lambda i: (i, 0)),
    )(x, m, b.reshape(1, f), dis.reshape(n, 1))


def _attention_pool(x, Wa):
    g = jnp.tanh(jnp.mean(x, axis=0) @ Wa)
    sig = jax.nn.sigmoid(x @ g)
    return x.T @ sig[:, None]


def _tensor_net(e1, e2, Wt, Wblock, bt):
    scoring = (e1.T @ Wt.reshape(F4, F4 * TN)).reshape(F4, TN)
    scoring = scoring.T @ e2
    combined = jnp.concatenate([e1, e2], axis=0)
    block = Wblock @ combined
    return jax.nn.relu(scoring + block + bt)


def kernel(features_1, features_2, edge_index_1, edge_index_2, W1, b1, W2, b2, W3, b3, Wl, bl, Wr, br, att, bias_g, Wa, Wt, Wblock, bt, Wf, bf, Ws, bs):
    loop = jnp.arange(N, dtype=jnp.int32)
    padi = (jnp.arange(PAD, dtype=jnp.int32) % 64)

    def edges(ei, off):
        s = jnp.concatenate([ei[0].astype(jnp.int32), loop, padi]) + off
        d = jnp.concatenate([ei[1].astype(jnp.int32), loop, padi + N])
        return s, d, d + off

    s1, d1, dt1 = edges(edge_index_1, 0)
    s2, d2, dt2 = edges(edge_index_2, N)
    s_r = jnp.stack([s1, s2]).reshape(NC, NS, NWIN, W)
    d_r = jnp.stack([d1, d2]).reshape(NC, NS, NWIN, W)
    dt_r = jnp.stack([dt1, dt2]).reshape(NC, NS, NWIN, W)

    degp = _sc_deg()(d_r)
    deg = degp[:, :N]
    dis = lax.rsqrt(deg)

    base1 = jax.random.key(1234 + 1)
    base2 = jax.random.key(1234 + 2)
    m1 = jnp.stack([_mask_scale(jax.random.fold_in(base1, 0), 0.8, (N, F1)),
                    _mask_scale(jax.random.fold_in(base2, 0), 0.8, (N, F1))])
    m2 = jnp.stack([_mask_scale(jax.random.fold_in(base1, 1), 0.5, (N, F2)),
                    _mask_scale(jax.random.fold_in(base2, 1), 0.5, (N, F2))])
    m3 = jnp.stack([_mask_scale(jax.random.fold_in(base1, 2), 0.5, (N, F3)),
                    _mask_scale(jax.random.fold_in(base2, 2), 0.5, (N, F3))])

    x = jnp.stack([features_1, features_2])
    dx = dis[:, :, None]

    def layer(h, Wk, bk, mk, f):
        tbl = (h * dx) @ Wk
        tbl = tbl.reshape(2 * N, f)
        p = _sc_agg(f)(tbl, s_r, d_r)[:, :N, :]
        z1 = _ew_apply(p[0], mk[0], bk, dis[0])
        z2 = _ew_apply(p[1], mk[1], bk, dis[1])
        return jnp.stack([z1, z2])

    h = layer(x, W1, b1, m1, F1)
    h = layer(h, W2, b2, m2, F2)
    h = layer(h, W3, b3, m3, F3)

    xln = h @ Wl + bl
    xrn = h @ Wr + br
    mh = jax.nn.leaky_relu(xln + xrn, negative_slope=0.2) @ att
    xlt = xln.reshape(NC * N, F4)
    xrt = jnp.concatenate(
        [xrn, mh[:, :, None], jnp.zeros((NC, N, FG - F4 - 1), jnp.float32)],
        axis=2).reshape(NC * N, FG)
    xrt = jnp.concatenate([xrt, jnp.zeros((64, FG), jnp.float32)], axis=0)

    XLs, XRm = _sc_gat_gather()(xlt, xrt, s_r, dt_r)
    Y = _edge_apply(XLs, XRm, att)
    accg = _sc_gat_scatter()(Y, d_r)
    gat = accg[:, :N, :F4] / (accg[:, :N, F4:F4 + 1] + 1e-16) + bias_g

    p1 = _attention_pool(gat[0], Wa)
    p2 = _attention_pool(gat[1], Wa)
    scores = _tensor_net(p1, p2, Wt, Wblock, bt).T
    hh = scores @ Wf.T + bf
    nrm = jnp.maximum(jnp.linalg.norm(hh, axis=1, keepdims=True), 1e-12)
    hh = hh / nrm
    return jax.nn.relu(hh @ Ws.T + bs)


# speedup vs baseline: 16.1654x; 1.1323x over previous
import functools

import jax
import jax.numpy as jnp
import numpy as np
from jax import lax
from jax.experimental import pallas as pl
from jax.experimental.pallas import tpu as pltpu
from jax.experimental.pallas import tpu_sc as plsc

N = 10000
E = 320000
D = 128
F1, F2, F3, F4 = 128, 64, 32, 32
TN = 16
BN = 16

NC, NS, LN = 2, 16, 16
W = 128
NWIN = 168
EPT = NWIN * W
EA_PAD = NS * EPT
EA = E + N
PAD = EA_PAD - EA
NROW = 10240
NBUF = 3

_mesh = plsc.VectorSubcoreMesh(core_axis_name="c", subcore_axis_name="s")


def _zero_rows(buf, nrow, f):
    z = jnp.zeros((LN,), jnp.float32)

    def body(r, _):
        for jj in range(f // LN):
            buf[0, r, pl.ds(jj * LN, LN)] = z
        return 0

    lax.fori_loop(0, nrow, body, 0, unroll=False)


def _sc_agg(f):
    spt = NROW // NS
    if f == 128:
        nbuf, slots, ch = 2, 2, 8
    else:
        nbuf, slots, ch = 3, 1, NWIN

    kw = {}
    if f != 128:
        kw["compiler_params"] = pltpu.CompilerParams(use_tc_tiling_on_sc=False)

    @functools.partial(
        pl.kernel,
        out_type=jax.ShapeDtypeStruct((NC, NROW, f), jnp.float32),
        mesh=_mesh,
        scratch_types=[
            pltpu.VMEM((slots, ch, W), jnp.int32),
            pltpu.VMEM((slots, ch, W), jnp.int32),
            pltpu.VMEM((nbuf, W, f), jnp.float32),
            pltpu.VMEM_SHARED((NROW, f), jnp.float32),
            pltpu.SemaphoreType.DMA((nbuf,)),
        ],
        **kw,
    )
    def k(tbl, s_hbm, d_hbm, out, sidx, didx, rows, acc, sem):
        c = lax.axis_index("c")
        t = lax.axis_index("s")
        pltpu.sync_copy(s_hbm.at[c, t, pl.ds(0, ch)], sidx.at[0])
        pltpu.sync_copy(d_hbm.at[c, t, pl.ds(0, ch)], didx.at[0])
        _zero_rows(rows, W, f)
        for kk in range(spt // W):
            pltpu.sync_copy(rows.at[0], acc.at[pl.ds(t * spt + kk * W, W)])
        plsc.subcore_barrier()

        def gref(w, b):
            slot = (w // ch) % slots
            return pltpu.make_async_copy(
                tbl.at[sidx.at[slot, w % ch]], rows.at[b], sem.at[b])

        for b in range(nbuf):
            gref(b, b).start()

        def body(i, _):
            for b in range(nbuf):
                w = i * nbuf + b
                gref(w, b).wait()
                slot = (w // ch) % slots
                pltpu.sync_copy(rows.at[b], acc.at[didx.at[slot, w % ch]],
                                add=True)
                wn = w + nbuf

                @pl.when(jnp.logical_and(wn < NWIN, wn % ch == 0))
                def _():
                    nc_ = wn // ch
                    ns_ = nc_ % slots
                    pltpu.sync_copy(s_hbm.at[c, t, pl.ds(nc_ * ch, ch)],
                                    sidx.at[ns_])
                    pltpu.sync_copy(d_hbm.at[c, t, pl.ds(nc_ * ch, ch)],
                                    didx.at[ns_])

                @pl.when(wn < NWIN)
                def _():
                    gref(wn, b).start()

            return 0

        lax.fori_loop(0, NWIN // nbuf, body, 0, unroll=False)
        plsc.subcore_barrier()
        pltpu.sync_copy(acc.at[pl.ds(t * spt, spt)], out.at[c, pl.ds(t * spt, spt)])

    return k


def _sc_deg():
    spt = NROW // NS

    @functools.partial(
        pl.kernel,
        out_type=jax.ShapeDtypeStruct((NC, NROW), jnp.float32),
        mesh=_mesh,
        scratch_types=[
            pltpu.VMEM((NWIN, W), jnp.int32),
            pltpu.VMEM((W,), jnp.float32),
            pltpu.VMEM_SHARED((NROW,), jnp.float32),
        ],
    )
    def k(d_hbm, out, didx, ones, acc):
        c = lax.axis_index("c")
        t = lax.axis_index("s")
        pltpu.sync_copy(d_hbm.at[c, t], didx)
        z = jnp.zeros((LN,), jnp.float32)
        for jj in range(W // LN):
            ones[pl.ds(jj * LN, LN)] = z
        for kk in range(spt // W):
            pltpu.sync_copy(ones, acc.at[pl.ds(t * spt + kk * W, W)])
        o = jnp.ones((LN,), jnp.float32)
        for jj in range(W // LN):
            ones[pl.ds(jj * LN, LN)] = o
        plsc.subcore_barrier()

        def body(w, _):
            pltpu.sync_copy(ones, acc.at[didx.at[w]], add=True)
            return 0

        lax.fori_loop(0, NWIN, body, 0, unroll=False)
        plsc.subcore_barrier()
        pltpu.sync_copy(acc.at[pl.ds(t * spt, spt)], out.at[c, pl.ds(t * spt, spt)])

    return k


FG = 48


def _vexp(x):
    x = jnp.minimum(jnp.maximum(x, -87.0), 88.0)
    y = x * 1.4426950408889634
    kf = (y + 0.5 * jnp.sign(y)).astype(jnp.int32)
    kff = kf.astype(jnp.float32)
    r = x - kff * 0.693359375
    r = r + kff * 2.1219444005469057e-4
    p = 1.0 / 720.0
    p = p * r + 1.0 / 120.0
    p = p * r + 1.0 / 24.0
    p = p * r + 1.0 / 6.0
    p = p * r + 0.5
    p = p * r + 1.0
    p = p * r + 1.0
    scale = plsc.bitcast(jnp.left_shift(kf + 127, 23), jnp.float32)
    return p * scale


def _sc_gat_fused():
    nbuf = 2
    spt = NROW // NS

    @functools.partial(
        pl.kernel,
        out_type=jax.ShapeDtypeStruct((NC, NROW, FG), jnp.float32),
        mesh=_mesh,
        scratch_types=[
            pltpu.VMEM((NWIN, W), jnp.int32),
            pltpu.VMEM((NWIN, W), jnp.int32),
            pltpu.VMEM((nbuf, W), jnp.int32),
            pltpu.VMEM((F4 * LN,), jnp.float32),
            pltpu.VMEM((nbuf, W, F4), jnp.float32),
            pltpu.VMEM((nbuf, W, FG), jnp.float32),
            pltpu.VMEM((nbuf, W, FG), jnp.float32),
            pltpu.VMEM_SHARED((NROW, FG), jnp.float32),
            pltpu.SemaphoreType.DMA((nbuf,)),
            pltpu.SemaphoreType.DMA((nbuf,)),
            pltpu.SemaphoreType.DMA((nbuf,)),
        ],
        compiler_params=pltpu.CompilerParams(use_tc_tiling_on_sc=False,
                                             needs_layout_passes=False),
    )
    def k(xlt, xrt, att_h, s_hbm, dt_hbm, out, sidx, dtidx, didx, attb,
          xbuf, rbuf, ybuf, acc, gsx, gsr, ssy):
        c = lax.axis_index("c")
        t = lax.axis_index("s")
        pltpu.sync_copy(s_hbm.at[c, t], sidx)
        pltpu.sync_copy(dt_hbm.at[c, t], dtidx)
        pltpu.sync_copy(att_h, attb)
        zv = jnp.zeros((LN,), jnp.float32)

        def zb(r, _):
            for sl in range(nbuf):
                for jj in range(FG // LN):
                    ybuf[sl, r, pl.ds(jj * LN, LN)] = zv
            return 0

        lax.fori_loop(0, W, zb, 0, unroll=False)
        for kk in range(spt // W):
            pltpu.sync_copy(ybuf.at[0], acc.at[pl.ds(t * spt + kk * W, W)])
        plsc.subcore_barrier()

        noff = c * N
        lane = lax.iota(jnp.int32, 16)

        def grefs(w, b):
            return (pltpu.make_async_copy(xlt.at[sidx.at[w]], xbuf.at[b],
                                          gsx.at[b]),
                    pltpu.make_async_copy(xrt.at[dtidx.at[w]], rbuf.at[b],
                                          gsr.at[b]))

        def sref(b):
            return pltpu.make_async_copy(ybuf.at[b], acc.at[didx.at[b]],
                                         ssy.at[b])

        for b in range(nbuf):
            for g in grefs(b, b):
                g.start()

        def body(i, _):
            for b in range(nbuf):
                w = i * nbuf + b
                for g in grefs(w, b):
                    g.wait()

                for jj in range(W // LN):
                    dv = dtidx[w, pl.ds(jj * LN, LN)]
                    didx[b, pl.ds(jj * LN, LN)] = dv - noff

                def grp(g_, _2):
                    el = lane + g_ * LN
                    alpha = jnp.zeros((LN,), jnp.float32)
                    for j in range(F4):
                        jv = jnp.full((LN,), j, jnp.int32)
                        xlj = plsc.load_gather(xbuf.at[b], [el, jv])
                        xrj = plsc.load_gather(rbuf.at[b], [el, jv])
                        tt = xlj + xrj
                        lr = jnp.maximum(tt, 0.2 * tt)
                        alpha = alpha + lr * attb[pl.ds(j * LN, LN)]
                    mhv = plsc.load_gather(rbuf.at[b],
                                           [el, jnp.full((LN,), F4, jnp.int32)])
                    av = _vexp(alpha - mhv)
                    plsc.store_scatter(ybuf.at[b],
                                       [el, jnp.full((LN,), F4, jnp.int32)], av)
                    for j in range(F4):
                        jv = jnp.full((LN,), j, jnp.int32)
                        xlj = plsc.load_gather(xbuf.at[b], [el, jv])
                        plsc.store_scatter(ybuf.at[b], [el, jv], av * xlj)
                    return 0

                lax.fori_loop(0, W // LN, grp, 0, unroll=False)
                pltpu.sync_copy(ybuf.at[b], acc.at[didx.at[b]], add=True)
                wn = w + nbuf

                @pl.when(wn < NWIN)
                def _():
                    for g in grefs(wn, b):
                        g.start()

            return 0

        lax.fori_loop(0, NWIN // nbuf, body, 0, unroll=False)
        plsc.subcore_barrier()
        pltpu.sync_copy(acc.at[pl.ds(t * spt, spt)], out.at[c, pl.ds(t * spt, spt)])

    return k


def _sc_gat_gather():
    nbuf = 2

    @functools.partial(
        pl.kernel,
        out_type=(jax.ShapeDtypeStruct((NC, EA_PAD, F4), jnp.float32),
                  jax.ShapeDtypeStruct((NC, EA_PAD, FG), jnp.float32)),
        mesh=_mesh,
        scratch_types=[
            pltpu.VMEM((NWIN, W), jnp.int32),
            pltpu.VMEM((NWIN, W), jnp.int32),
            pltpu.VMEM((nbuf, W, F4), jnp.float32),
            pltpu.VMEM((nbuf, W, FG), jnp.float32),
            pltpu.SemaphoreType.DMA((nbuf,)),
            pltpu.SemaphoreType.DMA((nbuf,)),
            pltpu.SemaphoreType.DMA((nbuf,)),
            pltpu.SemaphoreType.DMA((nbuf,)),
        ],
        compiler_params=pltpu.CompilerParams(use_tc_tiling_on_sc=False),
    )
    def k(xlt, xrt, s_hbm, dt_hbm, oxl, oxr, sidx, didx, xbuf, rbuf,
          gsx, gsr, wsx, wsr):
        c = lax.axis_index("c")
        t = lax.axis_index("s")
        pltpu.sync_copy(s_hbm.at[c, t], sidx)
        pltpu.sync_copy(dt_hbm.at[c, t], didx)

        def grefs(w, b):
            return (pltpu.make_async_copy(xlt.at[sidx.at[w]], xbuf.at[b],
                                          gsx.at[b]),
                    pltpu.make_async_copy(xrt.at[didx.at[w]], rbuf.at[b],
                                          gsr.at[b]))

        def wrefs(w, b):
            o = t * EPT + w * W
            return (pltpu.make_async_copy(xbuf.at[b], oxl.at[c, pl.ds(o, W)],
                                          wsx.at[b]),
                    pltpu.make_async_copy(rbuf.at[b], oxr.at[c, pl.ds(o, W)],
                                          wsr.at[b]))

        for b in range(nbuf):
            for g in grefs(b, b):
                g.start()

        def body(i, _):
            for b in range(nbuf):
                w = i * nbuf + b
                for g in grefs(w, b):
                    g.wait()
                for wr in wrefs(w, b):
                    wr.start()
                wn = w + nbuf

                @pl.when(wn < NWIN)
                def _():
                    for wr in wrefs(w, b):
                        wr.wait()
                    for g in grefs(wn, b):
                        g.start()

            return 0

        lax.fori_loop(0, NWIN // nbuf, body, 0, unroll=False)
        for b in range(nbuf):
            w = NWIN - nbuf + b
            for wr in wrefs(w, b):
                wr.wait()

    return k


def _sc_gat_scatter():
    nbuf = 3
    spt = NROW // NS

    @functools.partial(
        pl.kernel,
        out_type=jax.ShapeDtypeStruct((NC, NROW, FG), jnp.float32),
        mesh=_mesh,
        scratch_types=[
            pltpu.VMEM((NWIN, W), jnp.int32),
            pltpu.VMEM((nbuf, W, FG), jnp.float32),
            pltpu.VMEM_SHARED((NROW, FG), jnp.float32),
            pltpu.SemaphoreType.DMA((nbuf,)),
        ],
        compiler_params=pltpu.CompilerParams(use_tc_tiling_on_sc=False),
    )
    def k(y_hbm, d_hbm, out, didx, rows, acc, sem):
        c = lax.axis_index("c")
        t = lax.axis_index("s")
        pltpu.sync_copy(d_hbm.at[c, t], didx)
        _zero_rows(rows, W, FG)
        for kk in range(spt // W):
            pltpu.sync_copy(rows.at[0], acc.at[pl.ds(t * spt + kk * W, W)])
        plsc.subcore_barrier()

        def gref(w, b):
            return pltpu.make_async_copy(
                y_hbm.at[c, pl.ds(t * EPT + w * W, W)], rows.at[b], sem.at[b])

        for b in range(nbuf):
            gref(b, b).start()

        def body(i, _):
            for b in range(nbuf):
                w = i * nbuf + b
                gref(w, b).wait()
                pltpu.sync_copy(rows.at[b], acc.at[didx.at[w]], add=True)
                wn = w + nbuf

                @pl.when(wn < NWIN)
                def _():
                    gref(wn, b).start()

            return 0

        lax.fori_loop(0, NWIN // nbuf, body, 0, unroll=False)
        plsc.subcore_barrier()
        pltpu.sync_copy(acc.at[pl.ds(t * spt, spt)], out.at[c, pl.ds(t * spt, spt)])

    return k


def _edge_kernel(xl_ref, xr_ref, att_ref, o_ref):
    xl = xl_ref[0]
    xrm = xr_ref[0]
    xr = xrm[:, :F4]
    mh = xrm[:, F4:F4 + 1]
    tt = xl + xr
    lr = jnp.maximum(tt, 0.2 * tt)
    al = lax.dot_general(lr, att_ref[...], (((1,), (1,)), ((), ())),
                         preferred_element_type=jnp.float32)
    a = jnp.exp(al - mh)
    y = a * xl
    blk = xl.shape[0]
    o_ref[0] = jnp.concatenate([y, a, jnp.zeros((blk, FG - F4 - 1), jnp.float32)],
                               axis=1)


def _edge_apply(XLs, XRm, att):
    blk = 4096
    nb = EA_PAD // blk
    return pl.pallas_call(
        _edge_kernel,
        out_shape=jax.ShapeDtypeStruct((NC, EA_PAD, FG), jnp.float32),
        grid=(NC, nb),
        in_specs=[
            pl.BlockSpec((1, blk, F4), lambda c, i: (c, i, 0)),
            pl.BlockSpec((1, blk, FG), lambda c, i: (c, i, 0)),
            pl.BlockSpec((1, F4), lambda c, i: (0, 0)),
        ],
        out_specs=pl.BlockSpec((1, blk, FG), lambda c, i: (c, i, 0)),
    )(XLs, XRm, att.reshape(1, F4))


def _mask_scale(key, p, shape):
    keep = 1.0 - p
    b = jax.random.bernoulli(key, keep, shape)
    return jnp.where(b, jnp.float32(1.0) / jnp.float32(keep), jnp.float32(0.0))


def _ew_kernel(x_ref, m_ref, b_ref, dis_ref, o_ref):
    z = jax.nn.relu(x_ref[...] * dis_ref[...] + b_ref[...])
    o_ref[...] = z * m_ref[...]


def _ew_apply(x, m, b, dis):
    n, f = x.shape
    blk = 1000
    return pl.pallas_call(
        _ew_kernel,
        out_shape=jax.ShapeDtypeStruct((n, f), jnp.float32),
        grid=(n // blk,),
        in_specs=[
            pl.BlockSpec((blk, f), lambda i: (i, 0)),
            pl.BlockSpec((blk, f), lambda i: (i, 0)),
            pl.BlockSpec((1, f), lambda i: (0, 0)),
            pl.BlockSpec((blk, 1), lambda i: (i, 0)),
        ],
        out_specs=pl.BlockSpec((blk, f), lambda i: (i, 0)),
    )(x, m, b.reshape(1, f), dis.reshape(n, 1))


def _attention_pool(x, Wa):
    g = jnp.tanh(jnp.mean(x, axis=0) @ Wa)
    sig = jax.nn.sigmoid(x @ g)
    return x.T @ sig[:, None]


def _tensor_net(e1, e2, Wt, Wblock, bt):
    scoring = (e1.T @ Wt.reshape(F4, F4 * TN)).reshape(F4, TN)
    scoring = scoring.T @ e2
    combined = jnp.concatenate([e1, e2], axis=0)
    block = Wblock @ combined
    return jax.nn.relu(scoring + block + bt)


def kernel(features_1, features_2, edge_index_1, edge_index_2, W1, b1, W2, b2, W3, b3, Wl, bl, Wr, br, att, bias_g, Wa, Wt, Wblock, bt, Wf, bf, Ws, bs):
    loop = jnp.arange(N, dtype=jnp.int32)
    padi = (jnp.arange(PAD, dtype=jnp.int32) % 64)

    def edges(ei, off):
        s = jnp.concatenate([ei[0].astype(jnp.int32), loop, padi]) + off
        d = jnp.concatenate([ei[1].astype(jnp.int32), loop, padi + N])
        return s, d, d + off

    s1, d1, dt1 = edges(edge_index_1, 0)
    s2, d2, dt2 = edges(edge_index_2, N)
    s_r = jnp.stack([s1, s2]).reshape(NC, NS, NWIN, W)
    d_r = jnp.stack([d1, d2]).reshape(NC, NS, NWIN, W)
    dt_r = jnp.stack([dt1, dt2]).reshape(NC, NS, NWIN, W)

    degp = _sc_deg()(d_r)
    deg = degp[:, :N]
    dis = lax.rsqrt(deg)

    base1 = jax.random.key(1234 + 1)
    base2 = jax.random.key(1234 + 2)
    m1 = jnp.stack([_mask_scale(jax.random.fold_in(base1, 0), 0.8, (N, F1)),
                    _mask_scale(jax.random.fold_in(base2, 0), 0.8, (N, F1))])
    m2 = jnp.stack([_mask_scale(jax.random.fold_in(base1, 1), 0.5, (N, F2)),
                    _mask_scale(jax.random.fold_in(base2, 1), 0.5, (N, F2))])
    m3 = jnp.stack([_mask_scale(jax.random.fold_in(base1, 2), 0.5, (N, F3)),
                    _mask_scale(jax.random.fold_in(base2, 2), 0.5, (N, F3))])

    x = jnp.stack([features_1, features_2])
    dx = dis[:, :, None]

    def layer(h, Wk, bk, mk, f):
        tbl = (h * dx) @ Wk
        tbl = tbl.reshape(2 * N, f)
        p = _sc_agg(f)(tbl, s_r, d_r)[:, :N, :]
        z1 = _ew_apply(p[0], mk[0], bk, dis[0])
        z2 = _ew_apply(p[1], mk[1], bk, dis[1])
        return jnp.stack([z1, z2])

    h = layer(x, W1, b1, m1, F1)
    h = layer(h, W2, b2, m2, F2)
    h = layer(h, W3, b3, m3, F3)

    xln = h @ Wl + bl
    xrn = h @ Wr + br
    mh = jax.nn.leaky_relu(xln + xrn, negative_slope=0.2) @ att
    xlt = xln.reshape(NC * N, F4)
    xrt = jnp.concatenate(
        [xrn, mh[:, :, None], jnp.zeros((NC, N, FG - F4 - 1), jnp.float32)],
        axis=2).reshape(NC * N, FG)
    xrt = jnp.concatenate([xrt, jnp.zeros((64, FG), jnp.float32)], axis=0)

    attb = jnp.tile(att[:, None], (1, LN)).reshape(-1)
    accg = _sc_gat_fused()(xlt, xrt, attb, s_r, dt_r)
    gat = accg[:, :N, :F4] / (accg[:, :N, F4:F4 + 1] + 1e-16) + bias_g

    p1 = _attention_pool(gat[0], Wa)
    p2 = _attention_pool(gat[1], Wa)
    scores = _tensor_net(p1, p2, Wt, Wblock, bt).T
    hh = scores @ Wf.T + bf
    nrm = jnp.maximum(jnp.linalg.norm(hh, axis=1, keepdims=True), 1e-12)
    hh = hh / nrm
    return jax.nn.relu(hh @ Ws.T + bs)


# speedup vs baseline: 16.6333x; 1.0289x over previous
import functools

import jax
import jax.numpy as jnp
import numpy as np
from jax import lax
from jax.experimental import pallas as pl
from jax.experimental.pallas import tpu as pltpu
from jax.experimental.pallas import tpu_sc as plsc

N = 10000
E = 320000
D = 128
F1, F2, F3, F4 = 128, 64, 32, 32
TN = 16
BN = 16

NC, NS, LN = 2, 16, 16
W = 128
NWIN = 168
EPT = NWIN * W
EA_PAD = NS * EPT
EA = E + N
PAD = EA_PAD - EA
NROW = 10240
NBUF = 3

_mesh = plsc.VectorSubcoreMesh(core_axis_name="c", subcore_axis_name="s")


def _zero_rows(buf, nrow, f):
    z = jnp.zeros((LN,), jnp.float32)

    def body(r, _):
        for jj in range(f // LN):
            buf[0, r, pl.ds(jj * LN, LN)] = z
        return 0

    lax.fori_loop(0, nrow, body, 0, unroll=False)


def _sc_agg(f):
    spt = NROW // NS
    if f == 128:
        nbuf, slots, ch = 2, 2, 8
    else:
        nbuf, slots, ch = 3, 1, NWIN

    kw = {}
    if f != 128:
        kw["compiler_params"] = pltpu.CompilerParams(use_tc_tiling_on_sc=False)

    @functools.partial(
        pl.kernel,
        out_type=jax.ShapeDtypeStruct((NC, NROW, f), jnp.float32),
        mesh=_mesh,
        scratch_types=[
            pltpu.VMEM((slots, ch, W), jnp.int32),
            pltpu.VMEM((slots, ch, W), jnp.int32),
            pltpu.VMEM((nbuf, W, f), jnp.float32),
            pltpu.VMEM_SHARED((NROW, f), jnp.float32),
            pltpu.SemaphoreType.DMA((nbuf,)),
        ],
        **kw,
    )
    def k(tbl, s_hbm, d_hbm, out, sidx, didx, rows, acc, sem):
        c = lax.axis_index("c")
        t = lax.axis_index("s")
        pltpu.sync_copy(s_hbm.at[c, t, pl.ds(0, ch)], sidx.at[0])
        pltpu.sync_copy(d_hbm.at[c, t, pl.ds(0, ch)], didx.at[0])
        _zero_rows(rows, W, f)
        for kk in range(spt // W):
            pltpu.sync_copy(rows.at[0], acc.at[pl.ds(t * spt + kk * W, W)])
        plsc.subcore_barrier()

        def gref(w, b):
            slot = (w // ch) % slots
            return pltpu.make_async_copy(
                tbl.at[sidx.at[slot, w % ch]], rows.at[b], sem.at[b])

        for b in range(nbuf):
            gref(b, b).start()

        def body(i, _):
            for b in range(nbuf):
                w = i * nbuf + b
                gref(w, b).wait()
                slot = (w // ch) % slots
                pltpu.sync_copy(rows.at[b], acc.at[didx.at[slot, w % ch]],
                                add=True)
                wn = w + nbuf

                @pl.when(jnp.logical_and(wn < NWIN, wn % ch == 0))
                def _():
                    nc_ = wn // ch
                    ns_ = nc_ % slots
                    pltpu.sync_copy(s_hbm.at[c, t, pl.ds(nc_ * ch, ch)],
                                    sidx.at[ns_])
                    pltpu.sync_copy(d_hbm.at[c, t, pl.ds(nc_ * ch, ch)],
                                    didx.at[ns_])

                @pl.when(wn < NWIN)
                def _():
                    gref(wn, b).start()

            return 0

        lax.fori_loop(0, NWIN // nbuf, body, 0, unroll=False)
        plsc.subcore_barrier()
        pltpu.sync_copy(acc.at[pl.ds(t * spt, spt)], out.at[c, pl.ds(t * spt, spt)])

    return k


def _sc_deg():
    spt = NROW // NS

    @functools.partial(
        pl.kernel,
        out_type=jax.ShapeDtypeStruct((NC, NROW), jnp.float32),
        mesh=_mesh,
        scratch_types=[
            pltpu.VMEM((NWIN, W), jnp.int32),
            pltpu.VMEM((W,), jnp.float32),
            pltpu.VMEM_SHARED((NROW,), jnp.float32),
        ],
    )
    def k(d_hbm, out, didx, ones, acc):
        c = lax.axis_index("c")
        t = lax.axis_index("s")
        pltpu.sync_copy(d_hbm.at[c, t], didx)
        z = jnp.zeros((LN,), jnp.float32)
        for jj in range(W // LN):
            ones[pl.ds(jj * LN, LN)] = z
        for kk in range(spt // W):
            pltpu.sync_copy(ones, acc.at[pl.ds(t * spt + kk * W, W)])
        o = jnp.ones((LN,), jnp.float32)
        for jj in range(W // LN):
            ones[pl.ds(jj * LN, LN)] = o
        plsc.subcore_barrier()

        def body(w, _):
            pltpu.sync_copy(ones, acc.at[didx.at[w]], add=True)
            return 0

        lax.fori_loop(0, NWIN, body, 0, unroll=False)
        plsc.subcore_barrier()
        pltpu.sync_copy(acc.at[pl.ds(t * spt, spt)], out.at[c, pl.ds(t * spt, spt)])

    return k


FG = 48


def _vexp(x):
    x = jnp.minimum(jnp.maximum(x, -87.0), 88.0)
    y = x * 1.4426950408889634
    kf = (y + 0.5 * jnp.sign(y)).astype(jnp.int32)
    kff = kf.astype(jnp.float32)
    r = x - kff * 0.693359375
    r = r + kff * 2.1219444005469057e-4
    p = 1.0 / 720.0
    p = p * r + 1.0 / 120.0
    p = p * r + 1.0 / 24.0
    p = p * r + 1.0 / 6.0
    p = p * r + 0.5
    p = p * r + 1.0
    p = p * r + 1.0
    scale = plsc.bitcast(jnp.left_shift(kf + 127, 23), jnp.float32)
    return p * scale


def _sc_gat_fused():
    nbuf = 2
    spt = NROW // NS

    @functools.partial(
        pl.kernel,
        out_type=jax.ShapeDtypeStruct((NC, NROW, FG), jnp.float32),
        mesh=_mesh,
        scratch_types=[
            pltpu.VMEM((NWIN, W), jnp.int32),
            pltpu.VMEM((NWIN, W), jnp.int32),
            pltpu.VMEM((nbuf, W), jnp.int32),
            pltpu.VMEM((F4 * LN,), jnp.float32),
            pltpu.VMEM((nbuf, W, F4), jnp.float32),
            pltpu.VMEM((nbuf, W, FG), jnp.float32),
            pltpu.VMEM((nbuf, W, FG), jnp.float32),
            pltpu.VMEM_SHARED((NROW, FG), jnp.float32),
            pltpu.SemaphoreType.DMA((nbuf,)),
            pltpu.SemaphoreType.DMA((nbuf,)),
            pltpu.SemaphoreType.DMA((nbuf,)),
        ],
        compiler_params=pltpu.CompilerParams(use_tc_tiling_on_sc=False,
                                             needs_layout_passes=False),
    )
    def k(xlt, xrt, att_h, s_hbm, dt_hbm, out, sidx, dtidx, didx, attb,
          xbuf, rbuf, ybuf, acc, gsx, gsr, ssy):
        c = lax.axis_index("c")
        t = lax.axis_index("s")
        pltpu.sync_copy(s_hbm.at[c, t], sidx)
        pltpu.sync_copy(dt_hbm.at[c, t], dtidx)
        pltpu.sync_copy(att_h, attb)
        zv = jnp.zeros((LN,), jnp.float32)

        def zb(r, _):
            for sl in range(nbuf):
                for jj in range(FG // LN):
                    ybuf[sl, r, pl.ds(jj * LN, LN)] = zv
            return 0

        lax.fori_loop(0, W, zb, 0, unroll=False)
        for kk in range(spt // W):
            pltpu.sync_copy(ybuf.at[0], acc.at[pl.ds(t * spt + kk * W, W)])
        plsc.subcore_barrier()

        noff = c * N
        lane = lax.iota(jnp.int32, 16)

        def grefs(w, b):
            return (pltpu.make_async_copy(xlt.at[sidx.at[w]], xbuf.at[b],
                                          gsx.at[b]),
                    pltpu.make_async_copy(xrt.at[dtidx.at[w]], rbuf.at[b],
                                          gsr.at[b]))

        def sref(b):
            return pltpu.make_async_copy(ybuf.at[b], acc.at[didx.at[b]],
                                         ssy.at[b])

        for b in range(nbuf):
            for g in grefs(b, b):
                g.start()

        def body(i, _):
            for b in range(nbuf):
                w = i * nbuf + b
                for g in grefs(w, b):
                    g.wait()

                @pl.when(w >= nbuf)
                def _():
                    sref(b).wait()

                for jj in range(W // LN):
                    dv = dtidx[w, pl.ds(jj * LN, LN)]
                    didx[b, pl.ds(jj * LN, LN)] = dv - noff

                def grp(g2, _2):
                    for gsub in range(2):
                        el = lane + (g2 * 2 + gsub) * LN
                        acc4 = [jnp.zeros((LN,), jnp.float32) for _ in range(4)]
                        for j in range(F4):
                            jv = jnp.full((LN,), j, jnp.int32)
                            xlj = plsc.load_gather(xbuf.at[b], [el, jv])
                            xrj = plsc.load_gather(rbuf.at[b], [el, jv])
                            tt = xlj + xrj
                            lr = jnp.maximum(tt, 0.2 * tt)
                            acc4[j % 4] = acc4[j % 4] + lr * attb[pl.ds(j * LN, LN)]
                        alpha = (acc4[0] + acc4[1]) + (acc4[2] + acc4[3])
                        mhv = plsc.load_gather(
                            rbuf.at[b], [el, jnp.full((LN,), F4, jnp.int32)])
                        av = _vexp(alpha - mhv)
                        plsc.store_scatter(
                            ybuf.at[b], [el, jnp.full((LN,), F4, jnp.int32)], av)
                        for j in range(F4):
                            jv = jnp.full((LN,), j, jnp.int32)
                            xlj = plsc.load_gather(xbuf.at[b], [el, jv])
                            plsc.store_scatter(ybuf.at[b], [el, jv], av * xlj)
                    return 0

                lax.fori_loop(0, W // LN // 2, grp, 0, unroll=False)
                sref(b).start()
                wn = w + nbuf

                @pl.when(wn < NWIN)
                def _():
                    for g in grefs(wn, b):
                        g.start()

            return 0

        lax.fori_loop(0, NWIN // nbuf, body, 0, unroll=False)
        for b in range(nbuf):
            sref(b).wait()
        plsc.subcore_barrier()
        pltpu.sync_copy(acc.at[pl.ds(t * spt, spt)], out.at[c, pl.ds(t * spt, spt)])

    return k


def _sc_gat_gather():
    nbuf = 2

    @functools.partial(
        pl.kernel,
        out_type=(jax.ShapeDtypeStruct((NC, EA_PAD, F4), jnp.float32),
                  jax.ShapeDtypeStruct((NC, EA_PAD, FG), jnp.float32)),
        mesh=_mesh,
        scratch_types=[
            pltpu.VMEM((NWIN, W), jnp.int32),
            pltpu.VMEM((NWIN, W), jnp.int32),
            pltpu.VMEM((nbuf, W, F4), jnp.float32),
            pltpu.VMEM((nbuf, W, FG), jnp.float32),
            pltpu.SemaphoreType.DMA((nbuf,)),
            pltpu.SemaphoreType.DMA((nbuf,)),
            pltpu.SemaphoreType.DMA((nbuf,)),
            pltpu.SemaphoreType.DMA((nbuf,)),
        ],
        compiler_params=pltpu.CompilerParams(use_tc_tiling_on_sc=False),
    )
    def k(xlt, xrt, s_hbm, dt_hbm, oxl, oxr, sidx, didx, xbuf, rbuf,
          gsx, gsr, wsx, wsr):
        c = lax.axis_index("c")
        t = lax.axis_index("s")
        pltpu.sync_copy(s_hbm.at[c, t], sidx)
        pltpu.sync_copy(dt_hbm.at[c, t], didx)

        def grefs(w, b):
            return (pltpu.make_async_copy(xlt.at[sidx.at[w]], xbuf.at[b],
                                          gsx.at[b]),
                    pltpu.make_async_copy(xrt.at[didx.at[w]], rbuf.at[b],
                                          gsr.at[b]))

        def wrefs(w, b):
            o = t * EPT + w * W
            return (pltpu.make_async_copy(xbuf.at[b], oxl.at[c, pl.ds(o, W)],
                                          wsx.at[b]),
                    pltpu.make_async_copy(rbuf.at[b], oxr.at[c, pl.ds(o, W)],
                                          wsr.at[b]))

        for b in range(nbuf):
            for g in grefs(b, b):
                g.start()

        def body(i, _):
            for b in range(nbuf):
                w = i * nbuf + b
                for g in grefs(w, b):
                    g.wait()
                for wr in wrefs(w, b):
                    wr.start()
                wn = w + nbuf

                @pl.when(wn < NWIN)
                def _():
                    for wr in wrefs(w, b):
                        wr.wait()
                    for g in grefs(wn, b):
                        g.start()

            return 0

        lax.fori_loop(0, NWIN // nbuf, body, 0, unroll=False)
        for b in range(nbuf):
            w = NWIN - nbuf + b
            for wr in wrefs(w, b):
                wr.wait()

    return k


def _sc_gat_scatter():
    nbuf = 3
    spt = NROW // NS

    @functools.partial(
        pl.kernel,
        out_type=jax.ShapeDtypeStruct((NC, NROW, FG), jnp.float32),
        mesh=_mesh,
        scratch_types=[
            pltpu.VMEM((NWIN, W), jnp.int32),
            pltpu.VMEM((nbuf, W, FG), jnp.float32),
            pltpu.VMEM_SHARED((NROW, FG), jnp.float32),
            pltpu.SemaphoreType.DMA((nbuf,)),
        ],
        compiler_params=pltpu.CompilerParams(use_tc_tiling_on_sc=False),
    )
    def k(y_hbm, d_hbm, out, didx, rows, acc, sem):
        c = lax.axis_index("c")
        t = lax.axis_index("s")
        pltpu.sync_copy(d_hbm.at[c, t], didx)
        _zero_rows(rows, W, FG)
        for kk in range(spt // W):
            pltpu.sync_copy(rows.at[0], acc.at[pl.ds(t * spt + kk * W, W)])
        plsc.subcore_barrier()

        def gref(w, b):
            return pltpu.make_async_copy(
                y_hbm.at[c, pl.ds(t * EPT + w * W, W)], rows.at[b], sem.at[b])

        for b in range(nbuf):
            gref(b, b).start()

        def body(i, _):
            for b in range(nbuf):
                w = i * nbuf + b
                gref(w, b).wait()
                pltpu.sync_copy(rows.at[b], acc.at[didx.at[w]], add=True)
                wn = w + nbuf

                @pl.when(wn < NWIN)
                def _():
                    gref(wn, b).start()

            return 0

        lax.fori_loop(0, NWIN // nbuf, body, 0, unroll=False)
        plsc.subcore_barrier()
        pltpu.sync_copy(acc.at[pl.ds(t * spt, spt)], out.at[c, pl.ds(t * spt, spt)])

    return k


def _edge_kernel(xl_ref, xr_ref, att_ref, o_ref):
    xl = xl_ref[0]
    xrm = xr_ref[0]
    xr = xrm[:, :F4]
    mh = xrm[:, F4:F4 + 1]
    tt = xl + xr
    lr = jnp.maximum(tt, 0.2 * tt)
    al = lax.dot_general(lr, att_ref[...], (((1,), (1,)), ((), ())),
                         preferred_element_type=jnp.float32)
    a = jnp.exp(al - mh)
    y = a * xl
    blk = xl.shape[0]
    o_ref[0] = jnp.concatenate([y, a, jnp.zeros((blk, FG - F4 - 1), jnp.float32)],
                               axis=1)


def _edge_apply(XLs, XRm, att):
    blk = 4096
    nb = EA_PAD // blk
    return pl.pallas_call(
        _edge_kernel,
        out_shape=jax.ShapeDtypeStruct((NC, EA_PAD, FG), jnp.float32),
        grid=(NC, nb),
        in_specs=[
            pl.BlockSpec((1, blk, F4), lambda c, i: (c, i, 0)),
            pl.BlockSpec((1, blk, FG), lambda c, i: (c, i, 0)),
            pl.BlockSpec((1, F4), lambda c, i: (0, 0)),
        ],
        out_specs=pl.BlockSpec((1, blk, FG), lambda c, i: (c, i, 0)),
    )(XLs, XRm, att.reshape(1, F4))


def _mask_scale(key, p, shape):
    keep = 1.0 - p
    b = jax.random.bernoulli(key, keep, shape)
    return jnp.where(b, jnp.float32(1.0) / jnp.float32(keep), jnp.float32(0.0))


def _ew_kernel(x_ref, m_ref, b_ref, dis_ref, o_ref):
    z = jax.nn.relu(x_ref[...] * dis_ref[...] + b_ref[...])
    o_ref[...] = z * m_ref[...]


def _ew_apply(x, m, b, dis):
    n, f = x.shape
    blk = 1000
    return pl.pallas_call(
        _ew_kernel,
        out_shape=jax.ShapeDtypeStruct((n, f), jnp.float32),
        grid=(n // blk,),
        in_specs=[
            pl.BlockSpec((blk, f), lambda i: (i, 0)),
            pl.BlockSpec((blk, f), lambda i: (i, 0)),
            pl.BlockSpec((1, f), lambda i: (0, 0)),
            pl.BlockSpec((blk, 1), lambda i: (i, 0)),
        ],
        out_specs=pl.BlockSpec((blk, f), lambda i: (i, 0)),
    )(x, m, b.reshape(1, f), dis.reshape(n, 1))


def _attention_pool(x, Wa):
    g = jnp.tanh(jnp.mean(x, axis=0) @ Wa)
    sig = jax.nn.sigmoid(x @ g)
    return x.T @ sig[:, None]


def _tensor_net(e1, e2, Wt, Wblock, bt):
    scoring = (e1.T @ Wt.reshape(F4, F4 * TN)).reshape(F4, TN)
    scoring = scoring.T @ e2
    combined = jnp.concatenate([e1, e2], axis=0)
    block = Wblock @ combined
    return jax.nn.relu(scoring + block + bt)


def kernel(features_1, features_2, edge_index_1, edge_index_2, W1, b1, W2, b2, W3, b3, Wl, bl, Wr, br, att, bias_g, Wa, Wt, Wblock, bt, Wf, bf, Ws, bs):
    loop = jnp.arange(N, dtype=jnp.int32)
    padi = (jnp.arange(PAD, dtype=jnp.int32) % 64)

    def edges(ei, off):
        s = jnp.concatenate([ei[0].astype(jnp.int32), loop, padi]) + off
        d = jnp.concatenate([ei[1].astype(jnp.int32), loop, padi + N])
        return s, d, d + off

    s1, d1, dt1 = edges(edge_index_1, 0)
    s2, d2, dt2 = edges(edge_index_2, N)
    s_r = jnp.stack([s1, s2]).reshape(NC, NS, NWIN, W)
    d_r = jnp.stack([d1, d2]).reshape(NC, NS, NWIN, W)
    dt_r = jnp.stack([dt1, dt2]).reshape(NC, NS, NWIN, W)

    degp = _sc_deg()(d_r)
    deg = degp[:, :N]
    dis = lax.rsqrt(deg)

    base1 = jax.random.key(1234 + 1)
    base2 = jax.random.key(1234 + 2)
    m1 = jnp.stack([_mask_scale(jax.random.fold_in(base1, 0), 0.8, (N, F1)),
                    _mask_scale(jax.random.fold_in(base2, 0), 0.8, (N, F1))])
    m2 = jnp.stack([_mask_scale(jax.random.fold_in(base1, 1), 0.5, (N, F2)),
                    _mask_scale(jax.random.fold_in(base2, 1), 0.5, (N, F2))])
    m3 = jnp.stack([_mask_scale(jax.random.fold_in(base1, 2), 0.5, (N, F3)),
                    _mask_scale(jax.random.fold_in(base2, 2), 0.5, (N, F3))])

    x = jnp.stack([features_1, features_2])
    dx = dis[:, :, None]

    def layer(h, Wk, bk, mk, f):
        tbl = (h * dx) @ Wk
        tbl = tbl.reshape(2 * N, f)
        p = _sc_agg(f)(tbl, s_r, d_r)[:, :N, :]
        z1 = _ew_apply(p[0], mk[0], bk, dis[0])
        z2 = _ew_apply(p[1], mk[1], bk, dis[1])
        return jnp.stack([z1, z2])

    h = layer(x, W1, b1, m1, F1)
    h = layer(h, W2, b2, m2, F2)
    h = layer(h, W3, b3, m3, F3)

    xln = h @ Wl + bl
    xrn = h @ Wr + br
    mh = jax.nn.leaky_relu(xln + xrn, negative_slope=0.2) @ att
    xlt = xln.reshape(NC * N, F4)
    xrt = jnp.concatenate(
        [xrn, mh[:, :, None], jnp.zeros((NC, N, FG - F4 - 1), jnp.float32)],
        axis=2).reshape(NC * N, FG)
    xrt = jnp.concatenate([xrt, jnp.zeros((64, FG), jnp.float32)], axis=0)

    attb = jnp.tile(att[:, None], (1, LN)).reshape(-1)
    accg = _sc_gat_fused()(xlt, xrt, attb, s_r, dt_r)
    gat = accg[:, :N, :F4] / (accg[:, :N, F4:F4 + 1] + 1e-16) + bias_g

    p1 = _attention_pool(gat[0], Wa)
    p2 = _attention_pool(gat[1], Wa)
    scores = _tensor_net(p1, p2, Wt, Wblock, bt).T
    hh = scores @ Wf.T + bf
    nrm = jnp.maximum(jnp.linalg.norm(hh, axis=1, keepdims=True), 1e-12)
    hh = hh / nrm
    return jax.nn.relu(hh @ Ws.T + bs)
